# baseline (speedup 1.0000x reference)
import functools

import jax
import jax.numpy as jnp
from jax import lax
from jax.experimental import pallas as pl
from jax.experimental.pallas import tpu as pltpu
from jax.experimental.pallas import tpu_sc as plsc

VOCAB = 1000000
D = 64
T = 200
B = 4096

NC = 2
NS = 16
NW = NC * NS
ROWS = B // NW
NBUF = 2

_mesh = plsc.VectorSubcoreMesh(core_axis_name="c", subcore_axis_name="s")

_scratch = []
for _ in range(NBUF):
    _scratch += [
        pltpu.VMEM((T,), jnp.int32),
        pltpu.VMEM((T,), jnp.int32),
        pltpu.VMEM((T, 2 * D), jnp.float32),
        pltpu.VMEM((T, D), jnp.float32),
    ]
_scratch += [
    pltpu.VMEM((T, D), jnp.float32),
    pltpu.SemaphoreType.DMA((NBUF,)),
    pltpu.SemaphoreType.DMA((NBUF,)),
    pltpu.SemaphoreType.DMA((NBUF,)),
    pltpu.SemaphoreType.DMA((NBUF,)),
]


@functools.partial(
    pl.kernel,
    mesh=_mesh,
    out_type=jax.ShapeDtypeStruct((B, T, D), jnp.float32),
    compiler_params=pltpu.CompilerParams(needs_layout_passes=False),
    scratch_types=_scratch,
)
def _embed(xr_hbm, xh_hbm, tab_hbm, pos_hbm, out_hbm, *scr):
    xr = [scr[4 * b + 0] for b in range(NBUF)]
    xh = [scr[4 * b + 1] for b in range(NBUF)]
    rv = [scr[4 * b + 2] for b in range(NBUF)]
    ov = [scr[4 * b + 3] for b in range(NBUF)]
    pos_v, sem_r, sem_h, sem_g, sem_o = scr[4 * NBUF:]

    wid = lax.axis_index("s") * NC + lax.axis_index("c")
    row0 = pl.multiple_of(wid * ROWS, ROWS)
    pltpu.sync_copy(pos_hbm, pos_v)

    def group_body(g):
        for b in range(NBUF):
            pltpu.async_copy(xr_hbm.at[row0 + g + b], xr[b], sem_r.at[b])
            pltpu.async_copy(xh_hbm.at[row0 + g + b], xh[b], sem_h.at[b])
        for b in range(NBUF):
            pltpu.make_async_copy(xr_hbm.at[row0 + g + b], xr[b],
                                  sem_r.at[b]).wait()
            pltpu.async_copy(tab_hbm.at[xr[b]], rv[b], sem_g.at[b])
        for b in range(NBUF):
            pltpu.make_async_copy(xh_hbm.at[row0 + g + b], xh[b],
                                  sem_h.at[b]).wait()
            pltpu.make_async_copy(tab_hbm.at[xr[b]], rv[b],
                                  sem_g.at[b]).wait()

            def blk_body(blk, carry, b=b):
                hv = xh[b][pl.ds(blk * 16, 16)]
                for k in range(16):
                    i = blk * 16 + k
                    off = pl.multiple_of(hv[k], D)
                    for fg in range(D // 16):
                        sl = pl.ds(fg * 16, 16)
                        ov[b][i, sl] = (rv[b][i, pl.ds(off + fg * 16, 16)]
                                        + pos_v[i, sl])
                return carry

            lax.fori_loop(0, T // 16, blk_body, 0)
            hv = xh[b][pl.ds(T - 16, 16)]
            for k in range(8, 16):
                i = T - 16 + k
                off = pl.multiple_of(hv[k], D)
                for fg in range(D // 16):
                    sl = pl.ds(fg * 16, 16)
                    ov[b][i, sl] = (rv[b][i, pl.ds(off + fg * 16, 16)]
                                    + pos_v[i, sl])
            pltpu.async_copy(ov[b], out_hbm.at[row0 + g + b], sem_o.at[b])
        for b in range(NBUF):
            pltpu.make_async_copy(ov[b], out_hbm.at[row0 + g + b],
                                  sem_o.at[b]).wait()

    pl.loop(0, ROWS, step=NBUF)(group_body)


def kernel(x, text_embedding, positional_embedding):
    xi = x.astype(jnp.int32)
    tab = text_embedding.reshape(VOCAB // 2, 2 * D)
    return _embed(xi >> 1, (xi & 1) * D, tab, positional_embedding)


# speedup vs baseline: 1.1082x; 1.1082x over previous
import functools

import jax
import jax.numpy as jnp
from jax import lax
from jax.experimental import pallas as pl
from jax.experimental.pallas import tpu as pltpu
from jax.experimental.pallas import tpu_sc as plsc

VOCAB = 1000000
D = 64
T = 200
B = 4096

NC = 2
NS = 16
NW = NC * NS

ROWS = B // NW
NBUF = 2

_mesh = plsc.VectorSubcoreMesh(core_axis_name="c", subcore_axis_name="s")

_scratch = []
for _ in range(NBUF):
    _scratch += [
        pltpu.VMEM((T,), jnp.int32),
        pltpu.VMEM((T, 2 * D), jnp.float32),
        pltpu.VMEM((T * D,), jnp.float32),
    ]
_scratch += [
    pltpu.VMEM((T, D), jnp.float32),
    pltpu.SemaphoreType.DMA((NBUF,)),
    pltpu.SemaphoreType.DMA((NBUF,)),
    pltpu.SemaphoreType.DMA((NBUF,)),
]


@functools.partial(
    pl.kernel,
    mesh=_mesh,
    out_type=jax.ShapeDtypeStruct((B, T * D), jnp.float32),
    compiler_params=pltpu.CompilerParams(needs_layout_passes=False),
    scratch_types=_scratch,
)
def _embed(x_hbm, tab_hbm, pos_hbm, out_hbm, *scr):
    xi = [scr[3 * b + 0] for b in range(NBUF)]
    rv = [scr[3 * b + 1] for b in range(NBUF)]
    ov = [scr[3 * b + 2] for b in range(NBUF)]
    pos_v, sem_i, sem_g, sem_o = scr[3 * NBUF:]

    wid = lax.axis_index("s") * NC + lax.axis_index("c")
    row0 = pl.multiple_of(wid * ROWS, ROWS)
    pltpu.sync_copy(pos_hbm, pos_v)

    def group_body(g):
        for b in range(NBUF):
            pltpu.async_copy(x_hbm.at[row0 + g + b], xi[b], sem_i.at[b])
        for b in range(NBUF):
            pltpu.make_async_copy(x_hbm.at[row0 + g + b], xi[b],
                                  sem_i.at[b]).wait()
            pltpu.async_copy(tab_hbm.at[xi[b]], rv[b], sem_g.at[b])
        for b in range(NBUF):
            pltpu.make_async_copy(tab_hbm.at[xi[b]], rv[b],
                                  sem_g.at[b]).wait()

            def tok_body(i, carry, b=b):
                for fg in range(D // 16):
                    sl = pl.ds(fg * 16, 16)
                    ov[b][pl.ds(i * D + fg * 16, 16)] = (
                        rv[b][i, sl] + pos_v[i, sl])
                return carry

            lax.fori_loop(0, T, tok_body, 0, unroll=8)
            pltpu.async_copy(ov[b], out_hbm.at[row0 + g + b], sem_o.at[b])
        for b in range(NBUF):
            pltpu.make_async_copy(ov[b], out_hbm.at[row0 + g + b],
                                  sem_o.at[b]).wait()

    pl.loop(0, ROWS, step=NBUF)(group_body)


def kernel(x, text_embedding, positional_embedding):
    tab = jnp.pad(text_embedding, ((0, 0), (0, D)))
    out = _embed(x.astype(jnp.int32), tab, positional_embedding)
    return out.reshape(B, T, D)
